# baseline (speedup 1.0000x reference)
import functools

import jax
import jax.numpy as jnp
from jax import lax
from jax.experimental import pallas as pl
from jax.experimental.pallas import tpu as pltpu
from jax.experimental.pallas import tpu_sc as plsc

_F = 26
_D = 32
_B = 16384
_NCAT = 100000
_NROW = _F * _NCAT

_info = plsc.get_sparse_core_info()
_NC, _NS = _info.num_cores, _info.num_subcores

_HB = _B // 2


def _sc_body(xb_hbm, tlin_hbm, bias_hbm, out_hbm, band_v, idx_v, out_v, bias_v):
    c = lax.axis_index("s") * _NC + lax.axis_index("c")

    pltpu.sync_copy(bias_hbm, bias_v)

    def feature(f, carry):
        pltpu.sync_copy(xb_hbm.at[pl.ds(f * _B, _B)], idx_v)
        pltpu.sync_copy(tlin_hbm.at[pl.ds(c * _NROW + f * _NCAT, _NCAT)], band_v)

        q = jnp.zeros((16,), jnp.int32) + (f * _D + c)
        bias_splat = plsc.load_gather(bias_v, [q])

        def half(h, _):
            def step(j, _):
                iv = idx_v[pl.ds(h * _HB + j * 16, 16)]
                vals = plsc.load_gather(band_v, [iv])
                out_v[pl.ds(j * 16, 16)] = vals + bias_splat
                return 0

            lax.fori_loop(0, _HB // 16, step, 0)
            pltpu.sync_copy(
                out_v, out_hbm.at[pl.ds((f * _D + c) * _B + h * _HB, _HB)])
            return 0

        lax.fori_loop(0, 2, half, 0)
        return carry

    lax.fori_loop(0, _F, feature, jnp.int32(0))


@jax.jit
def _tokenize(xb, tlin, bias_flat):
    kern = functools.partial(
        pl.kernel,
        mesh=plsc.VectorSubcoreMesh(core_axis_name="c", subcore_axis_name="s"),
        out_type=jax.ShapeDtypeStruct((_F * _D * _B,), jnp.float32),
        scratch_types=[
            pltpu.VMEM((_NCAT,), jnp.float32),
            pltpu.VMEM((_B,), jnp.int32),
            pltpu.VMEM((_HB,), jnp.float32),
            pltpu.VMEM((_F * _D,), jnp.float32),
        ],
        compiler_params=pltpu.CompilerParams(needs_layout_passes=False),
    )(_sc_body)
    return kern(xb, tlin, bias_flat)


def kernel(x, table, bias):
    out = _tokenize(x.T.reshape(-1), table.T.reshape(-1), bias.reshape(-1))
    return out.reshape(_F, _D, _B).transpose(2, 0, 1)


# speedup vs baseline: 2.5660x; 2.5660x over previous
import functools

import jax
import jax.numpy as jnp
import numpy as np
from jax import lax
from jax.experimental import pallas as pl
from jax.experimental.pallas import tpu as pltpu
from jax.experimental.pallas import tpu_sc as plsc

_NUM_CATEGORIES = [100000] * 26
_F = len(_NUM_CATEGORIES)
_D = 32
_B = 16384
_BF = _B * _F
_NROW = sum(_NUM_CATEGORIES)

_info = plsc.get_sparse_core_info()
_NC, _NS = _info.num_cores, _info.num_subcores
_NW = _NC * _NS

_ALIGNED_ROWS = (_NROW // 128) * 128
_BW = 1024
_NBLK = _ALIGNED_ROWS // _BW
_TAIL = _NROW - _ALIGNED_ROWS

_RPW = _BF // _NW
_PR = 128
_NP = _RPW // _PR

_offsets_np = np.cumsum([0] + _NUM_CATEGORIES[:-1]).astype(np.int32)
_OFF2 = np.concatenate([_offsets_np, _offsets_np])
_BIAS_PAT = _F * _D


def _detile_body(tab_hbm, tail_hbm, out_hbm, blk_v, dst_v, tail_v):
    wid = lax.axis_index("s") * _NC + lax.axis_index("c")

    lanes = lax.iota(jnp.int32, 16)

    def block(t, carry):
        b = wid + t * _NW

        @pl.when(b < _NBLK)
        def _():
            col0 = b * _BW
            pltpu.sync_copy(tab_hbm.at[:, pl.ds(col0, _BW)], blk_v)

            def row(r, _):
                rv = jnp.zeros((16,), jnp.int32) + r
                dst_v[pl.ds(r * _D, 16)] = plsc.load_gather(blk_v, [lanes, rv])
                dst_v[pl.ds(r * _D + 16, 16)] = plsc.load_gather(blk_v, [lanes + 16, rv])
                return 0

            lax.fori_loop(0, _BW, row, 0)
            pltpu.sync_copy(dst_v, out_hbm.at[pl.ds(col0 * _D, _BW * _D)])

        return carry

    lax.fori_loop(0, (_NBLK + _NW - 1) // _NW, block, jnp.int32(0))

    @pl.when(wid == 0)
    def _():
        pltpu.sync_copy(tail_hbm, tail_v)
        pltpu.sync_copy(tail_v, out_hbm.at[pl.ds(_ALIGNED_ROWS * _D, _TAIL * _D)])


def _gather_body(x_hbm, off2_hbm, bias_hbm, table_hbm, out_hbm,
                 idx_all, rows, off2_v, bias2_v, sem):
    wid = lax.axis_index("s") * _NC + lax.axis_index("c")

    pltpu.sync_copy(off2_hbm, off2_v)
    pltpu.sync_copy(bias_hbm, bias2_v.at[pl.ds(0, _BIAS_PAT)])
    pltpu.sync_copy(bias_hbm, bias2_v.at[pl.ds(_BIAS_PAT, _BIAS_PAT)])

    row0 = wid * _NP
    pltpu.sync_copy(x_hbm.at[pl.ds(row0, _NP), :], idx_all)

    def off_row(r, q):
        for c in range(_PR // 16):
            chunk = idx_all[r, pl.ds(c * 16, 16)]
            idx_all[r, pl.ds(c * 16, 16)] = chunk + off2_v[pl.ds(q, 16)]
            q = q + 16
            q = jnp.where(q >= _F, q - _F, q)
        return q

    lax.fori_loop(0, _NP, off_row, jnp.int32(0))

    base = wid * _RPW

    def piece(i, carry):
        pltpu.async_copy(table_hbm.at[idx_all.at[i]], rows, sem).wait()

        qb0 = lax.rem(i * _PR, _F) * _D

        def bias_row(r, qb):
            rows[r, pl.ds(0, 16)] = rows[r, pl.ds(0, 16)] + bias2_v[pl.ds(qb, 16)]
            rows[r, pl.ds(16, 16)] = rows[r, pl.ds(16, 16)] + bias2_v[pl.ds(qb + 16, 16)]
            qb = qb + _D
            return jnp.where(qb >= _BIAS_PAT, qb - _BIAS_PAT, qb)

        lax.fori_loop(0, _PR, bias_row, qb0)

        pltpu.sync_copy(rows, out_hbm.at[pl.ds(base + i * _PR, _PR), :])
        return carry

    lax.fori_loop(0, _NP, piece, jnp.int32(0))


@jax.jit
def _tokenize(x2d, off2, bias_flat, table_t, tail_flat):
    mesh = plsc.VectorSubcoreMesh(core_axis_name="c", subcore_axis_name="s")

    detile = functools.partial(
        pl.kernel,
        mesh=mesh,
        out_type=jax.ShapeDtypeStruct((_NROW * _D,), jnp.float32),
        scratch_types=[
            pltpu.VMEM((_D, _BW), jnp.float32),
            pltpu.VMEM((_BW * _D,), jnp.float32),
            pltpu.VMEM((_TAIL * _D,), jnp.float32),
        ],
        compiler_params=pltpu.CompilerParams(needs_layout_passes=False),
    )(_detile_body)
    table_rm = detile(table_t, tail_flat).reshape(_NROW, _D)

    gather = functools.partial(
        pl.kernel,
        mesh=mesh,
        out_type=jax.ShapeDtypeStruct((_BF, _D), jnp.float32),
        scratch_types=[
            pltpu.VMEM((_NP, _PR), jnp.int32),
            pltpu.VMEM((_PR, _D), jnp.float32),
            pltpu.VMEM((2 * _F,), jnp.int32),
            pltpu.VMEM((2 * _BIAS_PAT,), jnp.float32),
            pltpu.SemaphoreType.DMA,
        ],
        compiler_params=pltpu.CompilerParams(use_tc_tiling_on_sc=False),
    )(_gather_body)
    return gather(x2d, off2, bias_flat, table_rm)


def kernel(x, table, bias):
    x2d = x.reshape(_BF // _PR, _PR)
    tail_flat = lax.slice(table, (_ALIGNED_ROWS, 0), (_NROW, _D)).reshape(-1)
    out = _tokenize(x2d, jnp.asarray(_OFF2), bias.reshape(-1), table.T, tail_flat)
    return out.reshape(_B, _F, _D)


# speedup vs baseline: 2.9324x; 1.1428x over previous
import functools

import jax
import jax.numpy as jnp
import numpy as np
from jax import lax
from jax.experimental import pallas as pl
from jax.experimental.pallas import tpu as pltpu
from jax.experimental.pallas import tpu_sc as plsc

_NUM_CATEGORIES = [100000] * 26
_F = len(_NUM_CATEGORIES)
_D = 32
_B = 16384
_BF = _B * _F
_NROW = sum(_NUM_CATEGORIES)

_info = plsc.get_sparse_core_info()
_NC, _NS = _info.num_cores, _info.num_subcores
_NW = _NC * _NS

_ALIGNED_ROWS = (_NROW // 128) * 128
_BW = 896
_NT = 92
_LASTCOL = _ALIGNED_ROWS - _BW
_TAIL = _NROW - _ALIGNED_ROWS

_RPW = _BF // _NW
_PR = 128
_NP = _RPW // _PR

_offsets_np = np.cumsum([0] + _NUM_CATEGORIES[:-1]).astype(np.int32)
_OFF2 = np.concatenate([_offsets_np, _offsets_np])
_BIAS_PAT = _F * _D


def _detile_body(tab_hbm, tail_hbm, out_hbm,
                 blk0, blk1, dst0, dst1, tail_v,
                 sin0, sin1, sout0, sout1):
    wid = lax.axis_index("s") * _NC + lax.axis_index("c")
    blk = [blk0, blk1]
    dst = [dst0, dst1]
    sin = [sin0, sin1]
    sout = [sout0, sout1]
    lanes = lax.iota(jnp.int32, 16)
    lanes16 = lanes + 16

    def colof(t):
        return jnp.minimum((wid + t * _NW) * _BW, _LASTCOL)

    def start_in(t, par):
        pltpu.async_copy(tab_hbm.at[:, pl.ds(colof(t), _BW)], blk[par], sin[par])

    def wait_in(t, par):
        pltpu.make_async_copy(
            tab_hbm.at[:, pl.ds(colof(t), _BW)], blk[par], sin[par]).wait()

    def start_out(t, par):
        pltpu.async_copy(
            dst[par], out_hbm.at[pl.ds(colof(t) * _D, _BW * _D)], sout[par])

    def wait_out(t, par):
        pltpu.make_async_copy(
            dst[par], out_hbm.at[pl.ds(colof(t) * _D, _BW * _D)], sout[par]).wait()

    start_in(0, 0)
    start_in(1, 1)

    def pair(p, carry):
        for par in (0, 1):
            t = p * 2 + par
            wait_in(t, par)

            @pl.when(t >= 2)
            def _():
                wait_out(t - 2, par)

            def rows2(r2, rv):
                r = r2 * 2
                for k in (0, 1):
                    dst[par][pl.ds((r + k) * _D, 16)] = plsc.load_gather(
                        blk[par], [lanes, rv])
                    dst[par][pl.ds((r + k) * _D + 16, 16)] = plsc.load_gather(
                        blk[par], [lanes16, rv])
                    rv = rv + 1
                return rv

            lax.fori_loop(0, _BW // 2, rows2, jnp.zeros((16,), jnp.int32))

            start_out(t, par)

            @pl.when(t + 2 < _NT)
            def _():
                start_in(t + 2, par)
        return carry

    lax.fori_loop(0, _NT // 2, pair, jnp.int32(0))
    wait_out(_NT - 2, 0)
    wait_out(_NT - 1, 1)

    @pl.when(wid == 0)
    def _():
        pltpu.sync_copy(tail_hbm, tail_v)
        pltpu.sync_copy(tail_v, out_hbm.at[pl.ds(_ALIGNED_ROWS * _D, _TAIL * _D)])


def _gather_body(x_hbm, off2_hbm, bias_hbm, table_hbm, out_hbm,
                 idx_all, rows0, rows1, off2_v, bias2_v,
                 sg0, sg1, so0, so1):
    wid = lax.axis_index("s") * _NC + lax.axis_index("c")
    rows = [rows0, rows1]
    sg = [sg0, sg1]
    so = [so0, so1]

    pltpu.sync_copy(off2_hbm, off2_v)
    pltpu.sync_copy(bias_hbm, bias2_v.at[pl.ds(0, _BIAS_PAT)])
    pltpu.sync_copy(bias_hbm, bias2_v.at[pl.ds(_BIAS_PAT, _BIAS_PAT)])

    row0 = wid * _NP
    pltpu.sync_copy(x_hbm.at[pl.ds(row0, _NP), :], idx_all)

    def off_row(r, q):
        for c in range(_PR // 16):
            chunk = idx_all[r, pl.ds(c * 16, 16)]
            idx_all[r, pl.ds(c * 16, 16)] = chunk + off2_v[pl.ds(q, 16)]
            q = q + 16
            q = jnp.where(q >= _F, q - _F, q)
        return q

    lax.fori_loop(0, _NP, off_row, jnp.int32(0))

    base = wid * _RPW

    def start_g(i, par):
        pltpu.async_copy(table_hbm.at[idx_all.at[i]], rows[par], sg[par])

    def wait_g(i, par):
        pltpu.make_async_copy(
            table_hbm.at[idx_all.at[i]], rows[par], sg[par]).wait()

    def start_o(i, par):
        pltpu.async_copy(
            rows[par], out_hbm.at[pl.ds(base + i * _PR, _PR), :], so[par])

    def wait_o(i, par):
        pltpu.make_async_copy(
            rows[par], out_hbm.at[pl.ds(base + i * _PR, _PR), :], so[par]).wait()

    start_g(0, 0)
    start_g(1, 1)

    def pair(p, carry):
        for par in (0, 1):
            i = p * 2 + par
            wait_g(i, par)

            @pl.when(i >= 2)
            def _():
                wait_o(i - 2, par)

            qb0 = lax.rem(i * _PR, _F) * _D

            def bias_row(r, qb):
                rows[par][r, pl.ds(0, 16)] = (
                    rows[par][r, pl.ds(0, 16)] + bias2_v[pl.ds(qb, 16)])
                rows[par][r, pl.ds(16, 16)] = (
                    rows[par][r, pl.ds(16, 16)] + bias2_v[pl.ds(qb + 16, 16)])
                qb = qb + _D
                return jnp.where(qb >= _BIAS_PAT, qb - _BIAS_PAT, qb)

            lax.fori_loop(0, _PR, bias_row, qb0)

            start_o(i, par)

            @pl.when(i + 2 < _NP)
            def _():
                start_g(i + 2, par)
        return carry

    lax.fori_loop(0, _NP // 2, pair, jnp.int32(0))
    wait_o(_NP - 2, 0)
    wait_o(_NP - 1, 1)


@jax.jit
def _tokenize(x2d, off2, bias_flat, table_t, tail_flat):
    mesh = plsc.VectorSubcoreMesh(core_axis_name="c", subcore_axis_name="s")

    detile = functools.partial(
        pl.kernel,
        mesh=mesh,
        out_type=jax.ShapeDtypeStruct((_NROW * _D,), jnp.float32),
        scratch_types=[
            pltpu.VMEM((_D, _BW), jnp.float32),
            pltpu.VMEM((_D, _BW), jnp.float32),
            pltpu.VMEM((_BW * _D,), jnp.float32),
            pltpu.VMEM((_BW * _D,), jnp.float32),
            pltpu.VMEM((_TAIL * _D,), jnp.float32),
            pltpu.SemaphoreType.DMA,
            pltpu.SemaphoreType.DMA,
            pltpu.SemaphoreType.DMA,
            pltpu.SemaphoreType.DMA,
        ],
        compiler_params=pltpu.CompilerParams(needs_layout_passes=False),
    )(_detile_body)
    table_rm = detile(table_t, tail_flat).reshape(_NROW, _D)

    gather = functools.partial(
        pl.kernel,
        mesh=mesh,
        out_type=jax.ShapeDtypeStruct((_BF, _D), jnp.float32),
        scratch_types=[
            pltpu.VMEM((_NP, _PR), jnp.int32),
            pltpu.VMEM((_PR, _D), jnp.float32),
            pltpu.VMEM((_PR, _D), jnp.float32),
            pltpu.VMEM((2 * _F,), jnp.int32),
            pltpu.VMEM((2 * _BIAS_PAT,), jnp.float32),
            pltpu.SemaphoreType.DMA,
            pltpu.SemaphoreType.DMA,
            pltpu.SemaphoreType.DMA,
            pltpu.SemaphoreType.DMA,
        ],
        compiler_params=pltpu.CompilerParams(use_tc_tiling_on_sc=False),
    )(_gather_body)
    return gather(x2d, off2, bias_flat, table_rm)


def kernel(x, table, bias):
    x2d = x.reshape(_BF // _PR, _PR)
    tail_flat = lax.slice(table, (_ALIGNED_ROWS, 0), (_NROW, _D)).reshape(-1)
    out = _tokenize(x2d, jnp.asarray(_OFF2), bias.reshape(-1), table.T, tail_flat)
    return out.reshape(_B, _F, _D)


# speedup vs baseline: 3.5285x; 1.2033x over previous
import functools

import jax
import jax.numpy as jnp
import numpy as np
from jax import lax
from jax.experimental import pallas as pl
from jax.experimental.pallas import tpu as pltpu
from jax.experimental.pallas import tpu_sc as plsc

_NUM_CATEGORIES = [100000] * 26
_F = len(_NUM_CATEGORIES)
_D = 32
_B = 16384
_BF = _B * _F
_NROW = sum(_NUM_CATEGORIES)

_info = plsc.get_sparse_core_info()
_NC, _NS = _info.num_cores, _info.num_subcores
_NW = _NC * _NS

_ALIGNED_ROWS = (_NROW // 128) * 128
_BW = 896
_NT = 92
_LASTCOL = _ALIGNED_ROWS - _BW
_TAIL = _NROW - _ALIGNED_ROWS

_RPW = _BF // _NW
_PR = 128
_NP = _RPW // _PR

_offsets_np = np.cumsum([0] + _NUM_CATEGORIES[:-1]).astype(np.int32)
_OFF2 = np.concatenate([_offsets_np, _offsets_np])
_BIAS_PAT = _F * _D


def _detile_body(tab_hbm, tail_hbm, out_hbm,
                 blk0, blk1, dst0, dst1, tail_v,
                 sin0, sin1, sout0, sout1):
    wid = lax.axis_index("s") * _NC + lax.axis_index("c")
    blk = [blk0, blk1]
    dst = [dst0, dst1]
    sin = [sin0, sin1]
    sout = [sout0, sout1]
    lanes32 = lax.iota(jnp.int32, 16) * _D

    def colof(t):
        return jnp.minimum((wid + t * _NW) * _BW, _LASTCOL)

    def start_in(t, par):
        pltpu.async_copy(tab_hbm.at[:, pl.ds(colof(t), _BW)], blk[par], sin[par])

    def wait_in(t, par):
        pltpu.make_async_copy(
            tab_hbm.at[:, pl.ds(colof(t), _BW)], blk[par], sin[par]).wait()

    def start_out(t, par):
        pltpu.async_copy(
            dst[par], out_hbm.at[pl.ds(colof(t) * _D, _BW * _D)], sout[par])

    def wait_out(t, par):
        pltpu.make_async_copy(
            dst[par], out_hbm.at[pl.ds(colof(t) * _D, _BW * _D)], sout[par]).wait()

    start_in(0, 0)
    start_in(1, 1)

    def pair(p, carry):
        for par in (0, 1):
            t = p * 2 + par
            wait_in(t, par)

            @pl.when(t >= 2)
            def _():
                wait_out(t - 2, par)

            def crow(c, _):
                idx0 = lanes32 + c

                def chunk4(k4, idx):
                    for kk in range(4):
                        v = blk[par][c, pl.ds((k4 * 4 + kk) * 16, 16)]
                        plsc.store_scatter(dst[par], [idx], v)
                        idx = idx + 16 * _D
                    return idx

                lax.fori_loop(0, _BW // 64, chunk4, idx0)
                return 0

            lax.fori_loop(0, _D, crow, 0)

            start_out(t, par)

            @pl.when(t + 2 < _NT)
            def _():
                start_in(t + 2, par)
        return carry

    lax.fori_loop(0, _NT // 2, pair, jnp.int32(0))
    wait_out(_NT - 2, 0)
    wait_out(_NT - 1, 1)

    @pl.when(wid == 0)
    def _():
        pltpu.sync_copy(tail_hbm, tail_v)
        pltpu.sync_copy(tail_v, out_hbm.at[pl.ds(_ALIGNED_ROWS * _D, _TAIL * _D)])


def _gather_body(x_hbm, off2_hbm, bias_hbm, table_hbm, out_hbm,
                 idx_all, rows0, rows1, off2_v, bias2_v,
                 sg0, sg1, so0, so1):
    wid = lax.axis_index("s") * _NC + lax.axis_index("c")
    rows = [rows0, rows1]
    sg = [sg0, sg1]
    so = [so0, so1]

    pltpu.sync_copy(off2_hbm, off2_v)
    pltpu.sync_copy(bias_hbm, bias2_v.at[pl.ds(0, _BIAS_PAT)])
    pltpu.sync_copy(bias_hbm, bias2_v.at[pl.ds(_BIAS_PAT, _BIAS_PAT)])

    row0 = wid * _NP
    pltpu.sync_copy(x_hbm.at[pl.ds(row0, _NP), :], idx_all)

    def off_row(r, q):
        for c in range(_PR // 16):
            chunk = idx_all[r, pl.ds(c * 16, 16)]
            idx_all[r, pl.ds(c * 16, 16)] = chunk + off2_v[pl.ds(q, 16)]
            q = q + 16
            q = jnp.where(q >= _F, q - _F, q)
        return q

    lax.fori_loop(0, _NP, off_row, jnp.int32(0))

    base = wid * _RPW

    def start_g(i, par):
        pltpu.async_copy(table_hbm.at[idx_all.at[i]], rows[par], sg[par])

    def wait_g(i, par):
        pltpu.make_async_copy(
            table_hbm.at[idx_all.at[i]], rows[par], sg[par]).wait()

    def start_o(i, par):
        pltpu.async_copy(
            rows[par], out_hbm.at[pl.ds(base + i * _PR, _PR), :], so[par])

    def wait_o(i, par):
        pltpu.make_async_copy(
            rows[par], out_hbm.at[pl.ds(base + i * _PR, _PR), :], so[par]).wait()

    start_g(0, 0)
    start_g(1, 1)

    def pair(p, carry):
        for par in (0, 1):
            i = p * 2 + par
            wait_g(i, par)

            @pl.when(i >= 2)
            def _():
                wait_o(i - 2, par)

            qb0 = lax.rem(i * _PR, _F) * _D

            def bias_row(r, qb):
                rows[par][r, pl.ds(0, 16)] = (
                    rows[par][r, pl.ds(0, 16)] + bias2_v[pl.ds(qb, 16)])
                rows[par][r, pl.ds(16, 16)] = (
                    rows[par][r, pl.ds(16, 16)] + bias2_v[pl.ds(qb + 16, 16)])
                qb = qb + _D
                return jnp.where(qb >= _BIAS_PAT, qb - _BIAS_PAT, qb)

            lax.fori_loop(0, _PR, bias_row, qb0)

            start_o(i, par)

            @pl.when(i + 2 < _NP)
            def _():
                start_g(i + 2, par)
        return carry

    lax.fori_loop(0, _NP // 2, pair, jnp.int32(0))
    wait_o(_NP - 2, 0)
    wait_o(_NP - 1, 1)


@jax.jit
def _tokenize(x2d, off2, bias_flat, table_t, tail_flat):
    mesh = plsc.VectorSubcoreMesh(core_axis_name="c", subcore_axis_name="s")

    detile = functools.partial(
        pl.kernel,
        mesh=mesh,
        out_type=jax.ShapeDtypeStruct((_NROW * _D,), jnp.float32),
        scratch_types=[
            pltpu.VMEM((_D, _BW), jnp.float32),
            pltpu.VMEM((_D, _BW), jnp.float32),
            pltpu.VMEM((_BW * _D,), jnp.float32),
            pltpu.VMEM((_BW * _D,), jnp.float32),
            pltpu.VMEM((_TAIL * _D,), jnp.float32),
            pltpu.SemaphoreType.DMA,
            pltpu.SemaphoreType.DMA,
            pltpu.SemaphoreType.DMA,
            pltpu.SemaphoreType.DMA,
        ],
        compiler_params=pltpu.CompilerParams(needs_layout_passes=False),
    )(_detile_body)
    table_rm = detile(table_t, tail_flat).reshape(_NROW, _D)

    gather = functools.partial(
        pl.kernel,
        mesh=mesh,
        out_type=jax.ShapeDtypeStruct((_BF, _D), jnp.float32),
        scratch_types=[
            pltpu.VMEM((_NP, _PR), jnp.int32),
            pltpu.VMEM((_PR, _D), jnp.float32),
            pltpu.VMEM((_PR, _D), jnp.float32),
            pltpu.VMEM((2 * _F,), jnp.int32),
            pltpu.VMEM((2 * _BIAS_PAT,), jnp.float32),
            pltpu.SemaphoreType.DMA,
            pltpu.SemaphoreType.DMA,
            pltpu.SemaphoreType.DMA,
            pltpu.SemaphoreType.DMA,
        ],
        compiler_params=pltpu.CompilerParams(use_tc_tiling_on_sc=False),
    )(_gather_body)
    return gather(x2d, off2, bias_flat, table_rm)


def kernel(x, table, bias):
    x2d = x.reshape(_BF // _PR, _PR)
    tail_flat = lax.slice(table, (_ALIGNED_ROWS, 0), (_NROW, _D)).reshape(-1)
    out = _tokenize(x2d, jnp.asarray(_OFF2), bias.reshape(-1), table.T, tail_flat)
    return out.reshape(_B, _F, _D)


# speedup vs baseline: 3.5783x; 1.0141x over previous
import functools

import jax
import jax.numpy as jnp
import numpy as np
from jax import lax
from jax.experimental import pallas as pl
from jax.experimental.pallas import tpu as pltpu
from jax.experimental.pallas import tpu_sc as plsc

_NUM_CATEGORIES = [100000] * 26
_F = len(_NUM_CATEGORIES)
_D = 32
_B = 16384
_BF = _B * _F
_NROW = sum(_NUM_CATEGORIES)

_info = plsc.get_sparse_core_info()
_NC, _NS = _info.num_cores, _info.num_subcores
_NW = _NC * _NS

_ALIGNED_ROWS = (_NROW // 128) * 128
_BW = 896
_NT = 92
_LASTCOL = _ALIGNED_ROWS - _BW
_TAIL = _NROW - _ALIGNED_ROWS

_RPW = _BF // _NW
_PR = 128
_NP = _RPW // _PR

_offsets_np = np.cumsum([0] + _NUM_CATEGORIES[:-1]).astype(np.int32)
_OFF2 = np.concatenate([_offsets_np, _offsets_np])
_BIAS_PAT = _F * _D


def _detile_body(tab_hbm, tail_hbm, out_hbm,
                 blk0, blk1, dst0, dst1, tail_v,
                 sin0, sin1, sout0, sout1):
    wid = lax.axis_index("s") * _NC + lax.axis_index("c")
    blk = [blk0, blk1]
    dst = [dst0, dst1]
    sin = [sin0, sin1]
    sout = [sout0, sout1]
    lanes32 = lax.iota(jnp.int32, 16) * _D

    def colof(t):
        return jnp.minimum((wid + t * _NW) * _BW, _LASTCOL)

    def start_in(t, par):
        pltpu.async_copy(tab_hbm.at[:, pl.ds(colof(t), _BW)], blk[par], sin[par])

    def wait_in(t, par):
        pltpu.make_async_copy(
            tab_hbm.at[:, pl.ds(colof(t), _BW)], blk[par], sin[par]).wait()

    def start_out(t, par):
        pltpu.async_copy(
            dst[par], out_hbm.at[pl.ds(colof(t) * _D, _BW * _D)], sout[par])

    def wait_out(t, par):
        pltpu.make_async_copy(
            dst[par], out_hbm.at[pl.ds(colof(t) * _D, _BW * _D)], sout[par]).wait()

    start_in(0, 0)
    start_in(1, 1)

    def pair(p, carry):
        for par in (0, 1):
            t = p * 2 + par
            wait_in(t, par)

            @pl.when(t >= 2)
            def _():
                wait_out(t - 2, par)

            def crow(c, _):
                base = lanes32 + c

                @plsc.parallel_loop(0, _BW // 16, unroll=8)
                def _chunk(k):
                    v = blk[par][c, pl.ds(k * 16, 16)]
                    plsc.store_scatter(dst[par], [base + k * (16 * _D)], v)

                return 0

            lax.fori_loop(0, _D, crow, 0)

            start_out(t, par)

            @pl.when(t + 2 < _NT)
            def _():
                start_in(t + 2, par)
        return carry

    lax.fori_loop(0, _NT // 2, pair, jnp.int32(0))
    wait_out(_NT - 2, 0)
    wait_out(_NT - 1, 1)

    @pl.when(wid == 0)
    def _():
        pltpu.sync_copy(tail_hbm, tail_v)
        pltpu.sync_copy(tail_v, out_hbm.at[pl.ds(_ALIGNED_ROWS * _D, _TAIL * _D)])


def _gather_body(x_hbm, off2_hbm, bias_hbm, table_hbm, out_hbm,
                 idx_all, rows0, rows1, off2_v, bias2_v,
                 sg0, sg1, so0, so1):
    wid = lax.axis_index("s") * _NC + lax.axis_index("c")
    rows = [rows0, rows1]
    sg = [sg0, sg1]
    so = [so0, so1]

    pltpu.sync_copy(off2_hbm, off2_v)
    pltpu.sync_copy(bias_hbm, bias2_v.at[pl.ds(0, _BIAS_PAT)])
    pltpu.sync_copy(bias_hbm, bias2_v.at[pl.ds(_BIAS_PAT, _BIAS_PAT)])

    row0 = wid * _NP
    pltpu.sync_copy(x_hbm.at[pl.ds(row0, _NP), :], idx_all)

    def off_row(r, q):
        for c in range(_PR // 16):
            chunk = idx_all[r, pl.ds(c * 16, 16)]
            idx_all[r, pl.ds(c * 16, 16)] = chunk + off2_v[pl.ds(q, 16)]
            q = q + 16
            q = jnp.where(q >= _F, q - _F, q)
        return q

    lax.fori_loop(0, _NP, off_row, jnp.int32(0))

    base = wid * _RPW

    def start_g(i, par):
        pltpu.async_copy(table_hbm.at[idx_all.at[i]], rows[par], sg[par])

    def wait_g(i, par):
        pltpu.make_async_copy(
            table_hbm.at[idx_all.at[i]], rows[par], sg[par]).wait()

    def start_o(i, par):
        pltpu.async_copy(
            rows[par], out_hbm.at[pl.ds(base + i * _PR, _PR), :], so[par])

    def wait_o(i, par):
        pltpu.make_async_copy(
            rows[par], out_hbm.at[pl.ds(base + i * _PR, _PR), :], so[par]).wait()

    start_g(0, 0)
    start_g(1, 1)

    def pair(p, carry):
        for par in (0, 1):
            i = p * 2 + par
            wait_g(i, par)

            @pl.when(i >= 2)
            def _():
                wait_o(i - 2, par)

            qb0 = lax.rem(i * _PR, _F) * _D

            def bias_row(r, qb):
                rows[par][r, pl.ds(0, 16)] = (
                    rows[par][r, pl.ds(0, 16)] + bias2_v[pl.ds(qb, 16)])
                rows[par][r, pl.ds(16, 16)] = (
                    rows[par][r, pl.ds(16, 16)] + bias2_v[pl.ds(qb + 16, 16)])
                qb = qb + _D
                return jnp.where(qb >= _BIAS_PAT, qb - _BIAS_PAT, qb)

            lax.fori_loop(0, _PR, bias_row, qb0)

            start_o(i, par)

            @pl.when(i + 2 < _NP)
            def _():
                start_g(i + 2, par)
        return carry

    lax.fori_loop(0, _NP // 2, pair, jnp.int32(0))
    wait_o(_NP - 2, 0)
    wait_o(_NP - 1, 1)


@jax.jit
def _tokenize(x2d, off2, bias_flat, table_t, tail_flat):
    mesh = plsc.VectorSubcoreMesh(core_axis_name="c", subcore_axis_name="s")

    detile = functools.partial(
        pl.kernel,
        mesh=mesh,
        out_type=jax.ShapeDtypeStruct((_NROW * _D,), jnp.float32),
        scratch_types=[
            pltpu.VMEM((_D, _BW), jnp.float32),
            pltpu.VMEM((_D, _BW), jnp.float32),
            pltpu.VMEM((_BW * _D,), jnp.float32),
            pltpu.VMEM((_BW * _D,), jnp.float32),
            pltpu.VMEM((_TAIL * _D,), jnp.float32),
            pltpu.SemaphoreType.DMA,
            pltpu.SemaphoreType.DMA,
            pltpu.SemaphoreType.DMA,
            pltpu.SemaphoreType.DMA,
        ],
        compiler_params=pltpu.CompilerParams(needs_layout_passes=False),
    )(_detile_body)
    table_rm = detile(table_t, tail_flat).reshape(_NROW, _D)

    gather = functools.partial(
        pl.kernel,
        mesh=mesh,
        out_type=jax.ShapeDtypeStruct((_BF, _D), jnp.float32),
        scratch_types=[
            pltpu.VMEM((_NP, _PR), jnp.int32),
            pltpu.VMEM((_PR, _D), jnp.float32),
            pltpu.VMEM((_PR, _D), jnp.float32),
            pltpu.VMEM((2 * _F,), jnp.int32),
            pltpu.VMEM((2 * _BIAS_PAT,), jnp.float32),
            pltpu.SemaphoreType.DMA,
            pltpu.SemaphoreType.DMA,
            pltpu.SemaphoreType.DMA,
            pltpu.SemaphoreType.DMA,
        ],
        compiler_params=pltpu.CompilerParams(use_tc_tiling_on_sc=False),
    )(_gather_body)
    return gather(x2d, off2, bias_flat, table_rm)


def kernel(x, table, bias):
    x2d = x.reshape(_BF // _PR, _PR)
    tail_flat = lax.slice(table, (_ALIGNED_ROWS, 0), (_NROW, _D)).reshape(-1)
    out = _tokenize(x2d, jnp.asarray(_OFF2), bias.reshape(-1), table.T, tail_flat)
    return out.reshape(_B, _F, _D)


# speedup vs baseline: 9.3986x; 2.6265x over previous
import functools

import jax
import jax.numpy as jnp
import numpy as np
from jax import lax
from jax.experimental import pallas as pl
from jax.experimental.pallas import tpu as pltpu
from jax.experimental.pallas import tpu_sc as plsc

_NUM_CATEGORIES = [100000] * 26
_F = len(_NUM_CATEGORIES)
_D = 32
_B = 16384
_BF = _B * _F
_NROW = sum(_NUM_CATEGORIES)

_info = plsc.get_sparse_core_info()
_NC, _NS = _info.num_cores, _info.num_subcores
_NW = _NC * _NS

_ALIGNED_ROWS = (_NROW // 128) * 128
_BW = 896
_NT = 92
_LASTCOL = _ALIGNED_ROWS - _BW
_TAIL = _NROW - _ALIGNED_ROWS

_RPW = _BF // _NW
_PR = 128
_NP = _RPW // _PR

_offsets_np = np.cumsum([0] + _NUM_CATEGORIES[:-1]).astype(np.int32)
_OFF2 = np.concatenate([_offsets_np, _offsets_np])
_BIAS_PAT = _F * _D


def _detile_body(tab_hbm, tail_hbm, out_hbm,
                 blk0, blk1, dst0, dst1, tail_v,
                 sin0, sin1, sout0, sout1):
    wid = lax.axis_index("s") * _NC + lax.axis_index("c")
    blk = [blk0, blk1]
    dst = [dst0, dst1]
    sin = [sin0, sin1]
    sout = [sout0, sout1]
    lanes = lax.iota(jnp.int32, 16)

    def colof(t):
        return jnp.minimum((wid + t * _NW) * _BW, _LASTCOL)

    def start_in(t, par):
        pltpu.async_copy(tab_hbm.at[:, pl.ds(colof(t), _BW)], blk[par], sin[par])

    def wait_in(t, par):
        pltpu.make_async_copy(
            tab_hbm.at[:, pl.ds(colof(t), _BW)], blk[par], sin[par]).wait()

    def start_out(t, par):
        pltpu.async_copy(
            dst[par], out_hbm.at[pl.ds(colof(t) * _D, _BW * _D)], sout[par])

    def wait_out(t, par):
        pltpu.make_async_copy(
            dst[par], out_hbm.at[pl.ds(colof(t) * _D, _BW * _D)], sout[par]).wait()

    start_in(0, 0)
    start_in(1, 1)

    def pair(p, carry):
        for par in (0, 1):
            t = p * 2 + par
            wait_in(t, par)

            @pl.when(t >= 2)
            def _():
                wait_out(t - 2, par)

            def rblock(rb, _):
                rvec = rb * 16 + lanes
                rvec32 = rvec * _D

                @plsc.parallel_loop(0, _D, unroll=8)
                def _diag(s):
                    cvec = (lanes + s) & (_D - 1)
                    v = plsc.load_gather(blk[par], [cvec, rvec])
                    plsc.store_scatter(dst[par], [rvec32 + cvec], v)

                return 0

            lax.fori_loop(0, _BW // 16, rblock, 0)

            start_out(t, par)

            @pl.when(t + 2 < _NT)
            def _():
                start_in(t + 2, par)
        return carry

    lax.fori_loop(0, _NT // 2, pair, jnp.int32(0))
    wait_out(_NT - 2, 0)
    wait_out(_NT - 1, 1)

    @pl.when(wid == 0)
    def _():
        pltpu.sync_copy(tail_hbm, tail_v)
        pltpu.sync_copy(tail_v, out_hbm.at[pl.ds(_ALIGNED_ROWS * _D, _TAIL * _D)])


def _gather_body(x_hbm, off2_hbm, bias_hbm, table_hbm, out_hbm,
                 idx_all, rows0, rows1, off2_v, bias2_v,
                 sg0, sg1, so0, so1):
    wid = lax.axis_index("s") * _NC + lax.axis_index("c")
    rows = [rows0, rows1]
    sg = [sg0, sg1]
    so = [so0, so1]

    pltpu.sync_copy(off2_hbm, off2_v)
    pltpu.sync_copy(bias_hbm, bias2_v.at[pl.ds(0, _BIAS_PAT)])
    pltpu.sync_copy(bias_hbm, bias2_v.at[pl.ds(_BIAS_PAT, _BIAS_PAT)])

    row0 = wid * _NP
    pltpu.sync_copy(x_hbm.at[pl.ds(row0, _NP), :], idx_all)

    def off_row(r, q):
        for c in range(_PR // 16):
            chunk = idx_all[r, pl.ds(c * 16, 16)]
            idx_all[r, pl.ds(c * 16, 16)] = chunk + off2_v[pl.ds(q, 16)]
            q = q + 16
            q = jnp.where(q >= _F, q - _F, q)
        return q

    lax.fori_loop(0, _NP, off_row, jnp.int32(0))

    base = wid * _RPW

    def start_g(i, par):
        pltpu.async_copy(table_hbm.at[idx_all.at[i]], rows[par], sg[par])

    def wait_g(i, par):
        pltpu.make_async_copy(
            table_hbm.at[idx_all.at[i]], rows[par], sg[par]).wait()

    def start_o(i, par):
        pltpu.async_copy(
            rows[par], out_hbm.at[pl.ds(base + i * _PR, _PR), :], so[par])

    def wait_o(i, par):
        pltpu.make_async_copy(
            rows[par], out_hbm.at[pl.ds(base + i * _PR, _PR), :], so[par]).wait()

    start_g(0, 0)
    start_g(1, 1)

    def pair(p, carry):
        for par in (0, 1):
            i = p * 2 + par
            wait_g(i, par)

            @pl.when(i >= 2)
            def _():
                wait_o(i - 2, par)

            qb0 = lax.rem(i * _PR, _F) * _D

            def bias_row(r, qb):
                rows[par][r, pl.ds(0, 16)] = (
                    rows[par][r, pl.ds(0, 16)] + bias2_v[pl.ds(qb, 16)])
                rows[par][r, pl.ds(16, 16)] = (
                    rows[par][r, pl.ds(16, 16)] + bias2_v[pl.ds(qb + 16, 16)])
                qb = qb + _D
                return jnp.where(qb >= _BIAS_PAT, qb - _BIAS_PAT, qb)

            lax.fori_loop(0, _PR, bias_row, qb0)

            start_o(i, par)

            @pl.when(i + 2 < _NP)
            def _():
                start_g(i + 2, par)
        return carry

    lax.fori_loop(0, _NP // 2, pair, jnp.int32(0))
    wait_o(_NP - 2, 0)
    wait_o(_NP - 1, 1)


@jax.jit
def _tokenize(x2d, off2, bias_flat, table_t, tail_flat):
    mesh = plsc.VectorSubcoreMesh(core_axis_name="c", subcore_axis_name="s")

    detile = functools.partial(
        pl.kernel,
        mesh=mesh,
        out_type=jax.ShapeDtypeStruct((_NROW * _D,), jnp.float32),
        scratch_types=[
            pltpu.VMEM((_D, _BW), jnp.float32),
            pltpu.VMEM((_D, _BW), jnp.float32),
            pltpu.VMEM((_BW * _D,), jnp.float32),
            pltpu.VMEM((_BW * _D,), jnp.float32),
            pltpu.VMEM((_TAIL * _D,), jnp.float32),
            pltpu.SemaphoreType.DMA,
            pltpu.SemaphoreType.DMA,
            pltpu.SemaphoreType.DMA,
            pltpu.SemaphoreType.DMA,
        ],
        compiler_params=pltpu.CompilerParams(needs_layout_passes=False),
    )(_detile_body)
    table_rm = detile(table_t, tail_flat).reshape(_NROW, _D)

    gather = functools.partial(
        pl.kernel,
        mesh=mesh,
        out_type=jax.ShapeDtypeStruct((_BF, _D), jnp.float32),
        scratch_types=[
            pltpu.VMEM((_NP, _PR), jnp.int32),
            pltpu.VMEM((_PR, _D), jnp.float32),
            pltpu.VMEM((_PR, _D), jnp.float32),
            pltpu.VMEM((2 * _F,), jnp.int32),
            pltpu.VMEM((2 * _BIAS_PAT,), jnp.float32),
            pltpu.SemaphoreType.DMA,
            pltpu.SemaphoreType.DMA,
            pltpu.SemaphoreType.DMA,
            pltpu.SemaphoreType.DMA,
        ],
        compiler_params=pltpu.CompilerParams(use_tc_tiling_on_sc=False),
    )(_gather_body)
    return gather(x2d, off2, bias_flat, table_rm)


def kernel(x, table, bias):
    x2d = x.reshape(_BF // _PR, _PR)
    tail_flat = lax.slice(table, (_ALIGNED_ROWS, 0), (_NROW, _D)).reshape(-1)
    out = _tokenize(x2d, jnp.asarray(_OFF2), bias.reshape(-1), table.T, tail_flat)
    return out.reshape(_B, _F, _D)
